# baseline (speedup 1.0000x reference)
import functools

import jax
import jax.numpy as jnp
from jax import lax
from jax.experimental import pallas as pl
from jax.experimental.pallas import tpu as pltpu
from jax.experimental.pallas import tpu_sc as plsc

_BATCH = 16384
_N = 256
_L = 16
_NG = _N // _L
_NC = 2
_NS = 16
_NW = _NC * _NS
_RPW = _BATCH // _NW
_CH = 64
_NCHUNK = _RPW // _CH


def _sc_body(x_hbm, w_hbm, no_hbm, out_hbm, no_v, w_v, xin_v, xout_v):
    wid = lax.axis_index("s") * _NC + lax.axis_index("c")
    base = wid * _RPW * _N

    pltpu.sync_copy(no_hbm, no_v)
    pltpu.sync_copy(w_hbm, w_v)

    def chunk_body(ci, _):
        off0 = base + ci * (_CH * _N)
        pltpu.sync_copy(x_hbm.at[pl.ds(off0, _CH * _N)], xin_v)
        for g in range(_NG):
            no_g = no_v[pl.ds(g * _L, _L)]
            w_g = w_v[pl.ds(g * _L, _L)]

            def row_body(r, carry, no_g=no_g, w_g=w_g):
                idx, off = carry
                vals = plsc.load_gather(xin_v, [idx])
                xout_v[pl.ds(off, _L)] = vals * w_g
                return idx + _N, off + _N

            lax.fori_loop(0, _CH, row_body, (no_g, jnp.int32(g * _L)))
        pltpu.sync_copy(xout_v, out_hbm.at[pl.ds(off0, _CH * _N)])
        return 0

    lax.fori_loop(0, _NCHUNK, chunk_body, 0)


@functools.partial(jax.jit, static_argnames=())
def _run(xf, weights, node_order):
    mesh = plsc.VectorSubcoreMesh(core_axis_name="c", subcore_axis_name="s")
    k = functools.partial(
        pl.kernel,
        mesh=mesh,
        out_type=jax.ShapeDtypeStruct((_BATCH * _N,), jnp.float32),
        compiler_params=pltpu.CompilerParams(needs_layout_passes=False),
        scratch_types=[
            pltpu.VMEM((_N,), jnp.int32),
            pltpu.VMEM((_N,), jnp.float32),
            pltpu.VMEM((_CH * _N,), jnp.float32),
            pltpu.VMEM((_CH * _N,), jnp.float32),
        ],
    )(_sc_body)
    return k(xf, weights, node_order)


def kernel(x, weights, node_order):
    out = _run(x.reshape(-1), weights, node_order)
    return out.reshape(_BATCH, _N)


# speedup vs baseline: 1.1022x; 1.1022x over previous
import functools

import jax
import jax.numpy as jnp
from jax import lax
from jax.experimental import pallas as pl
from jax.experimental.pallas import tpu as pltpu
from jax.experimental.pallas import tpu_sc as plsc

_BATCH = 16384
_N = 256
_L = 16
_NG = _N // _L
_NC = 2
_NS = 16
_NW = _NC * _NS
_RPW = _BATCH // _NW
_CH = 64
_NCHUNK = _RPW // _CH
_CHW = _CH * _N


def _sc_body(x_hbm, w_hbm, no_hbm, out_hbm,
             no_v, w_v, xin0, xin1, xout0, xout1,
             sin0, sin1, sout0, sout1):
    wid = lax.axis_index("s") * _NC + lax.axis_index("c")
    base = wid * _RPW * _N

    pltpu.sync_copy(no_hbm, no_v)
    pltpu.sync_copy(w_hbm, w_v)
    no_g = [no_v[pl.ds(g * _L, _L)] for g in range(_NG)]
    w_g = [w_v[pl.ds(g * _L, _L)] for g in range(_NG)]

    xin = (xin0, xin1)
    xout = (xout0, xout1)
    sin = (sin0, sin1)
    sout = (sout0, sout1)

    in_dma = [None, None]
    out_dma = [None, None]
    in_dma[0] = pltpu.async_copy(x_hbm.at[pl.ds(base, _CHW)], xin[0], sin[0])

    for ci in range(_NCHUNK):
        b = ci % 2
        off = base + ci * _CHW
        in_dma[b].wait()
        if ci + 1 < _NCHUNK:
            nb = (ci + 1) % 2
            in_dma[nb] = pltpu.async_copy(
                x_hbm.at[pl.ds(off + _CHW, _CHW)], xin[nb], sin[nb])
        if out_dma[b] is not None:
            out_dma[b].wait()

        def row_body(r, _, xi=xin[b], xo=xout[b]):
            rbase = r * _N
            rvec = jnp.full((_L,), rbase, jnp.int32)
            for g in range(_NG):
                vals = plsc.load_gather(xi, [no_g[g] + rvec])
                xo[pl.ds(rbase + g * _L, _L)] = vals * w_g[g]
            return 0

        lax.fori_loop(0, _CH, row_body, 0)
        out_dma[b] = pltpu.async_copy(xout[b], out_hbm.at[pl.ds(off, _CHW)],
                                      sout[b])

    out_dma[0].wait()
    out_dma[1].wait()


@jax.jit
def _run(xf, weights, node_order):
    mesh = plsc.VectorSubcoreMesh(core_axis_name="c", subcore_axis_name="s")
    k = functools.partial(
        pl.kernel,
        mesh=mesh,
        out_type=jax.ShapeDtypeStruct((_BATCH * _N,), jnp.float32),
        compiler_params=pltpu.CompilerParams(needs_layout_passes=False),
        scratch_types=[
            pltpu.VMEM((_N,), jnp.int32),
            pltpu.VMEM((_N,), jnp.float32),
            pltpu.VMEM((_CHW,), jnp.float32),
            pltpu.VMEM((_CHW,), jnp.float32),
            pltpu.VMEM((_CHW,), jnp.float32),
            pltpu.VMEM((_CHW,), jnp.float32),
            pltpu.SemaphoreType.DMA,
            pltpu.SemaphoreType.DMA,
            pltpu.SemaphoreType.DMA,
            pltpu.SemaphoreType.DMA,
        ],
    )(_sc_body)
    return k(xf, weights, node_order)


def kernel(x, weights, node_order):
    out = _run(x.reshape(-1), weights, node_order)
    return out.reshape(_BATCH, _N)


# speedup vs baseline: 1.4129x; 1.2819x over previous
import functools

import jax
import jax.numpy as jnp
from jax import lax
from jax.experimental import pallas as pl
from jax.experimental.pallas import tpu as pltpu
from jax.experimental.pallas import tpu_sc as plsc

_BATCH = 16384
_N = 256
_L = 16
_NG = _N // _L
_NC = 2
_NS = 16
_NW = _NC * _NS
_RPW = _BATCH // _NW
_CH = 64
_NCHUNK = _RPW // _CH
_CHW = _CH * _N


def _sc_body(x_hbm, w_hbm, no_hbm, out_hbm,
             no_v, w_v, xin0, xin1, xout0, xout1,
             sin0, sin1, sout0, sout1):
    wid = lax.axis_index("s") * _NC + lax.axis_index("c")
    base = wid * _RPW * _N

    pltpu.sync_copy(no_hbm, no_v)
    pltpu.sync_copy(w_hbm, w_v)
    no_g = [no_v[pl.ds(g * _L, _L)] for g in range(_NG)]
    w_g = [w_v[pl.ds(g * _L, _L)] for g in range(_NG)]

    xin = (xin0, xin1)
    xout = (xout0, xout1)
    sin = (sin0, sin1)
    sout = (sout0, sout1)

    in_dma = [None, None]
    out_dma = [None, None]
    in_dma[0] = pltpu.async_copy(x_hbm.at[pl.ds(base, _CHW)], xin[0], sin[0])

    for ci in range(_NCHUNK):
        b = ci % 2
        off = base + ci * _CHW
        in_dma[b].wait()
        if ci + 1 < _NCHUNK:
            nb = (ci + 1) % 2
            in_dma[nb] = pltpu.async_copy(
                x_hbm.at[pl.ds(off + _CHW, _CHW)], xin[nb], sin[nb])
        if out_dma[b] is not None:
            out_dma[b].wait()

        def make_row_body(xi, xo):
            def row_body(r):
                rbase = r * _N
                rvec = jnp.full((_L,), rbase, jnp.int32)
                for g in range(_NG):
                    vals = plsc.load_gather(xi, [no_g[g] + rvec])
                    xo[pl.ds(rbase + g * _L, _L)] = vals * w_g[g]
            return row_body

        plsc.parallel_loop(0, _CH, unroll=2)(make_row_body(xin[b], xout[b]))
        out_dma[b] = pltpu.async_copy(xout[b], out_hbm.at[pl.ds(off, _CHW)],
                                      sout[b])

    out_dma[0].wait()
    out_dma[1].wait()


@jax.jit
def _run(xf, weights, node_order):
    mesh = plsc.VectorSubcoreMesh(core_axis_name="c", subcore_axis_name="s")
    k = functools.partial(
        pl.kernel,
        mesh=mesh,
        out_type=jax.ShapeDtypeStruct((_BATCH * _N,), jnp.float32),
        compiler_params=pltpu.CompilerParams(needs_layout_passes=False),
        scratch_types=[
            pltpu.VMEM((_N,), jnp.int32),
            pltpu.VMEM((_N,), jnp.float32),
            pltpu.VMEM((_CHW,), jnp.float32),
            pltpu.VMEM((_CHW,), jnp.float32),
            pltpu.VMEM((_CHW,), jnp.float32),
            pltpu.VMEM((_CHW,), jnp.float32),
            pltpu.SemaphoreType.DMA,
            pltpu.SemaphoreType.DMA,
            pltpu.SemaphoreType.DMA,
            pltpu.SemaphoreType.DMA,
        ],
    )(_sc_body)
    return k(xf, weights, node_order)


def kernel(x, weights, node_order):
    out = _run(x.reshape(-1), weights, node_order)
    return out.reshape(_BATCH, _N)


# speedup vs baseline: 2.6762x; 1.8941x over previous
import functools

import jax
import jax.numpy as jnp
from jax import lax
from jax.experimental import pallas as pl
from jax.experimental.pallas import tpu as pltpu
from jax.experimental.pallas import tpu_sc as plsc

_BATCH = 16384
_N = 256
_L = 16
_NG = _N // _L
_NC = 2
_NS = 16
_NW = _NC * _NS
_RPW = _BATCH // _NW
_CH = 64
_NCHUNK = _RPW // _CH


def _sc_body(x_hbm, w_hbm, no_hbm, out_hbm,
             no_v, w_v, xin0, xin1, xout0, xout1,
             sin0, sin1, sout0, sout1):
    wid = lax.axis_index("s") * _NC + lax.axis_index("c")
    base = wid * _RPW

    pltpu.sync_copy(no_hbm, no_v)
    pltpu.sync_copy(w_hbm, w_v)
    no_g = [no_v[pl.ds(g * _L, _L)] for g in range(_NG)]
    w_g = [w_v[pl.ds(g * _L, _L)] for g in range(_NG)]

    xin = (xin0, xin1)
    xout = (xout0, xout1)
    sin = (sin0, sin1)
    sout = (sout0, sout1)

    in_dma = [None, None]
    out_dma = [None, None]
    in_dma[0] = pltpu.async_copy(x_hbm.at[pl.ds(base, _CH)], xin[0], sin[0])

    for ci in range(_NCHUNK):
        b = ci % 2
        row0 = base + ci * _CH
        in_dma[b].wait()
        if ci + 1 < _NCHUNK:
            nb = (ci + 1) % 2
            in_dma[nb] = pltpu.async_copy(
                x_hbm.at[pl.ds(row0 + _CH, _CH)], xin[nb], sin[nb])
        if out_dma[b] is not None:
            out_dma[b].wait()

        def make_row_body(xi, xo):
            def row_body(r):
                rvec = jnp.full((_L,), r, jnp.int32)
                for g in range(_NG):
                    vals = plsc.load_gather(xi, [rvec, no_g[g]])
                    xo[r, pl.ds(g * _L, _L)] = vals * w_g[g]
            return row_body

        plsc.parallel_loop(0, _CH, unroll=2)(make_row_body(xin[b], xout[b]))
        out_dma[b] = pltpu.async_copy(xout[b], out_hbm.at[pl.ds(row0, _CH)],
                                      sout[b])

    out_dma[0].wait()
    out_dma[1].wait()


@jax.jit
def _run(x, weights, node_order):
    mesh = plsc.VectorSubcoreMesh(core_axis_name="c", subcore_axis_name="s")
    k = functools.partial(
        pl.kernel,
        mesh=mesh,
        out_type=jax.ShapeDtypeStruct((_BATCH, _N), jnp.float32),
        compiler_params=pltpu.CompilerParams(needs_layout_passes=False),
        scratch_types=[
            pltpu.VMEM((_N,), jnp.int32),
            pltpu.VMEM((_N,), jnp.float32),
            pltpu.VMEM((_CH, _N), jnp.float32),
            pltpu.VMEM((_CH, _N), jnp.float32),
            pltpu.VMEM((_CH, _N), jnp.float32),
            pltpu.VMEM((_CH, _N), jnp.float32),
            pltpu.SemaphoreType.DMA,
            pltpu.SemaphoreType.DMA,
            pltpu.SemaphoreType.DMA,
            pltpu.SemaphoreType.DMA,
        ],
    )(_sc_body)
    return k(x, weights, node_order)


def kernel(x, weights, node_order):
    return _run(x, weights, node_order)


# speedup vs baseline: 3.2099x; 1.1994x over previous
import functools

import jax
import jax.numpy as jnp
from jax import lax
from jax.experimental import pallas as pl
from jax.experimental.pallas import tpu as pltpu
from jax.experimental.pallas import tpu_sc as plsc

_BATCH = 16384
_N = 256
_L = 16
_NG = _N // _L
_NC = 2
_NS = 16
_NW = _NC * _NS
_RPW = _BATCH // _NW
_CH = 64
_NCHUNK = _RPW // _CH
_NPAIR = _NCHUNK // 2


def _sc_body(x_hbm, w_hbm, no_hbm, out_hbm,
             no_v, w_v, xin0, xin1, xout0, xout1,
             sin0, sin1, sout0, sout1):
    wid = lax.axis_index("s") * _NC + lax.axis_index("c")
    base = wid * _RPW
    last = base + _RPW - _CH

    pltpu.sync_copy(no_hbm, no_v)
    pltpu.sync_copy(w_hbm, w_v)
    no_g = [no_v[pl.ds(g * _L, _L)] for g in range(_NG)]
    w_g = [w_v[pl.ds(g * _L, _L)] for g in range(_NG)]

    def start_in(buf, sem, row0):
        pltpu.async_copy(x_hbm.at[pl.ds(row0, _CH)], buf, sem)

    def wait_in(buf, sem):
        pltpu.make_async_copy(x_hbm.at[pl.ds(0, _CH)], buf, sem).wait()

    def start_out(buf, sem, row0):
        pltpu.async_copy(buf, out_hbm.at[pl.ds(row0, _CH)], sem)

    def wait_out(buf, sem):
        pltpu.make_async_copy(buf, out_hbm.at[pl.ds(0, _CH)], sem).wait()

    def compute(xi, xo):
        def row_body(r):
            rvec = jnp.full((_L,), r, jnp.int32)
            for g in range(_NG):
                vals = plsc.load_gather(xi, [rvec, no_g[g]])
                xo[r, pl.ds(g * _L, _L)] = vals * w_g[g]
        plsc.parallel_loop(0, _CH, unroll=1)(row_body)

    start_in(xin0, sin0, base)

    def pair_body(i, _):
        c0 = base + (2 * i) * _CH
        c1 = c0 + _CH
        start_in(xin1, sin1, c1)
        wait_in(xin0, sin0)

        @pl.when(i > 0)
        def _():
            wait_out(xout0, sout0)

        compute(xin0, xout0)
        start_out(xout0, sout0, c0)
        start_in(xin0, sin0, jnp.minimum(c1 + _CH, last))
        wait_in(xin1, sin1)

        @pl.when(i > 0)
        def _():
            wait_out(xout1, sout1)

        compute(xin1, xout1)
        start_out(xout1, sout1, c1)
        return 0

    lax.fori_loop(0, _NPAIR, pair_body, 0)

    wait_in(xin0, sin0)
    wait_out(xout0, sout0)
    wait_out(xout1, sout1)


@jax.jit
def _run(x, weights, node_order):
    mesh = plsc.VectorSubcoreMesh(core_axis_name="c", subcore_axis_name="s")
    k = functools.partial(
        pl.kernel,
        mesh=mesh,
        out_type=jax.ShapeDtypeStruct((_BATCH, _N), jnp.float32),
        compiler_params=pltpu.CompilerParams(needs_layout_passes=False),
        scratch_types=[
            pltpu.VMEM((_N,), jnp.int32),
            pltpu.VMEM((_N,), jnp.float32),
            pltpu.VMEM((_CH, _N), jnp.float32),
            pltpu.VMEM((_CH, _N), jnp.float32),
            pltpu.VMEM((_CH, _N), jnp.float32),
            pltpu.VMEM((_CH, _N), jnp.float32),
            pltpu.SemaphoreType.DMA,
            pltpu.SemaphoreType.DMA,
            pltpu.SemaphoreType.DMA,
            pltpu.SemaphoreType.DMA,
        ],
    )(_sc_body)
    return k(x, weights, node_order)


def kernel(x, weights, node_order):
    return _run(x, weights, node_order)
